# baseline (speedup 1.0000x reference)
import math
import numpy as np
import jax
import jax.numpy as jnp
from jax.experimental import pallas as pl
from jax.experimental.pallas import tpu as pltpu

HID = 2048; NH = 16; DK = 128; DR = 64; DV = 128; DC = 512; DCP = 1536
INH = 8; IHD = 128; ITOPK = 8
HALF = DR // 2
MB = 256
QB = 256
HPG = 2
SCALE = 1.0 / math.sqrt(DK + DR)
BF = jnp.bfloat16
F32 = jnp.float32


def _rope_tables(S):
    inv = 1.0 / (10000.0 ** (np.arange(0, DR, 2)[: DR // 2].astype(np.float32) / DR))
    t = np.arange(S, dtype=np.float32)
    f = np.outer(t, inv)
    return jnp.asarray(np.cos(f), dtype=F32), jnp.asarray(np.sin(f), dtype=F32)


def _nt(a, b):
    return jax.lax.dot_general(a, b, (((1,), (1,)), ((), ())),
                               preferred_element_type=F32)


def _ka_body(x_ref, wc_ref, wcp_ref, wkr_ref, wigt_ref, wqc_ref, wqr_ref,
             wkc_ref, wv_ref, cos_ref, sin_ref,
             qc_ref, qr_ref, kc_ref, v_ref, kr_ref, gt_ref):
    xb = x_ref[...]
    xb16 = xb.astype(BF)
    gt_ref[...] = _nt(wigt_ref[...], xb)
    co = cos_ref[...]
    si = sin_ref[...]
    y = jnp.dot(xb16, wkr_ref[...], preferred_element_type=F32)
    a = y[:, :HALF]
    b = y[:, HALF:]
    kr_ref[...] = jnp.concatenate(
        [a * co - b * si, a * si + b * co], axis=1).astype(BF)
    c16 = jnp.dot(xb16, wc_ref[...], preferred_element_type=F32).astype(BF)
    cp16 = jnp.dot(xb16, wcp_ref[...], preferred_element_type=F32).astype(BF)
    qc = jnp.dot(cp16, wqc_ref[...], preferred_element_type=F32) * SCALE
    qc_ref[...] = qc.astype(BF)
    yq = jnp.dot(cp16, wqr_ref[...], preferred_element_type=F32) * SCALE
    parts = []
    for h in range(NH):
        qa = yq[:, h * DR:h * DR + HALF]
        qb = yq[:, h * DR + HALF:(h + 1) * DR]
        parts += [qa * co - qb * si, qa * si + qb * co]
    qr_ref[...] = jnp.concatenate(parts, axis=1).astype(BF)
    kc_ref[...] = jnp.dot(c16, wkc_ref[...], preferred_element_type=F32).astype(BF)
    v_ref[...] = jnp.dot(c16, wv_ref[...], preferred_element_type=F32).astype(BF)


def _ka(x2, W_c, W_cp, W_kr_d, W_igate_t, W_qc, W_qr_d, W_kc, W_v, cos_t, sin_t):
    S = x2.shape[0]
    const = lambda i: (0, 0)
    row = lambda i: (i, 0)
    return pl.pallas_call(
        _ka_body,
        grid=(S // MB,),
        in_specs=[
            pl.BlockSpec((MB, HID), row),
            pl.BlockSpec((HID, DC), const),
            pl.BlockSpec((HID, DCP), const),
            pl.BlockSpec((HID, DR), const),
            pl.BlockSpec((INH, HID), const),
            pl.BlockSpec((DCP, NH * DK), const),
            pl.BlockSpec((DCP, NH * DR), const),
            pl.BlockSpec((DC, NH * DK), const),
            pl.BlockSpec((DC, NH * DV), const),
            pl.BlockSpec((MB, HALF), row),
            pl.BlockSpec((MB, HALF), row),
        ],
        out_specs=[
            pl.BlockSpec((MB, NH * DK), row),
            pl.BlockSpec((MB, NH * DR), row),
            pl.BlockSpec((MB, NH * DK), row),
            pl.BlockSpec((MB, NH * DV), row),
            pl.BlockSpec((MB, DR), row),
            pl.BlockSpec((INH, MB), lambda i: (0, i)),
        ],
        out_shape=[
            jax.ShapeDtypeStruct((S, NH * DK), BF),
            jax.ShapeDtypeStruct((S, NH * DR), BF),
            jax.ShapeDtypeStruct((S, NH * DK), BF),
            jax.ShapeDtypeStruct((S, NH * DV), BF),
            jax.ShapeDtypeStruct((S, DR), BF),
            jax.ShapeDtypeStruct((INH, S), F32),
        ],
    )(x2, W_c, W_cp, W_kr_d, W_igate_t, W_qc, W_qr_d, W_kc, W_v, cos_t, sin_t)


def _kb_body(qc_ref, qr_ref, kc_ref, kr_ref, v_ref, o_ref):
    S = qc_ref.shape[0]
    nq = S // QB
    kr = kr_ref[...]
    for hh in range(HPG):
        kc = kc_ref[:, hh * DK:(hh + 1) * DK]
        v = v_ref[:, hh * DV:(hh + 1) * DV]
        for i in range(nq):
            qc = qc_ref[i * QB:(i + 1) * QB, hh * DK:(hh + 1) * DK]
            qr = qr_ref[i * QB:(i + 1) * QB, hh * DR:(hh + 1) * DR]
            m = jnp.full((QB, 1), -1e30, F32)
            l = jnp.zeros((QB, 1), F32)
            acc = jnp.zeros((QB, DV), F32)
            rows = i * QB + jax.lax.broadcasted_iota(jnp.int32, (QB, QB), 0)
            for j in range(i + 1):
                s = _nt(qc, kc[j * QB:(j + 1) * QB, :])
                s = s + _nt(qr, kr[j * QB:(j + 1) * QB, :])
                if j == i:
                    cols = j * QB + jax.lax.broadcasted_iota(jnp.int32, (QB, QB), 1)
                    s = jnp.where(cols > rows, -1e30, s)
                mb = jnp.max(s, axis=1, keepdims=True)
                m_new = jnp.maximum(m, mb)
                p = jnp.exp(s - m_new)
                corr = jnp.exp(m - m_new)
                l = l * corr + jnp.sum(p, axis=1, keepdims=True)
                acc = acc * corr + jnp.dot(p.astype(BF), v[j * QB:(j + 1) * QB, :],
                                           preferred_element_type=F32)
                m = m_new
            o_ref[i * QB:(i + 1) * QB, hh * DV:(hh + 1) * DV] = (acc / l).astype(BF)


def _kb(qc, qr, kc, kr, v):
    S = qc.shape[0]
    return pl.pallas_call(
        _kb_body,
        grid=(NH // HPG,),
        in_specs=[
            pl.BlockSpec((S, HPG * DK), lambda h: (0, h)),
            pl.BlockSpec((S, HPG * DR), lambda h: (0, h)),
            pl.BlockSpec((S, HPG * DK), lambda h: (0, h)),
            pl.BlockSpec((S, DR), lambda h: (0, 0)),
            pl.BlockSpec((S, HPG * DV), lambda h: (0, h)),
        ],
        out_specs=pl.BlockSpec((S, HPG * DV), lambda h: (0, h)),
        out_shape=jax.ShapeDtypeStruct((S, NH * DV), BF),
    )(qc, qr, kc, kr, v)


def _kc_body(gt_ref, x_ref, wip_ref, wsq_ref, wsk_ref, wsv_ref, wio_ref,
             a_ref, wo_ref, o_ref, bias_ref):
    i = pl.program_id(0)

    @pl.when(i == 0)
    def _indexer():
        g = gt_ref[...]
        Sn = g.shape[1]
        col = jax.lax.broadcasted_iota(jnp.int32, g.shape, 1)
        gg = g
        oh_rows = []
        for _ in range(ITOPK):
            mx = jnp.max(gg, axis=1, keepdims=True)
            amx = jnp.min(jnp.where(gg >= mx, col, Sn), axis=1)
            oh_rows.append((col == amx[:, None]).astype(BF))
            gg = jnp.where(col == amx[:, None], -jnp.inf, gg)
        r64 = jax.lax.broadcasted_iota(jnp.int32, (INH * ITOPK, INH), 0)
        h8 = jax.lax.broadcasted_iota(jnp.int32, (INH * ITOPK, INH), 1)
        oh = jnp.zeros((INH * ITOPK, Sn), F32)
        for t in range(ITOPK):
            pt = jnp.where(r64 == h8 * ITOPK + t, 1.0, 0.0).astype(BF)
            oh = oh + jnp.dot(pt, oh_rows[t], preferred_element_type=F32)
        x_sel = jnp.dot(oh.astype(BF), x_ref[...],
                        preferred_element_type=F32).astype(BF)
        sel = []
        for h in range(INH):
            sel.append(jnp.dot(x_sel[h * ITOPK:(h + 1) * ITOPK, :],
                               wip_ref[:, h * IHD:(h + 1) * IHD],
                               preferred_element_type=F32))
        s64 = jnp.concatenate(sel, axis=0)
        sq = jnp.dot(s64, wsq_ref[...], preferred_element_type=F32)
        sk = jnp.dot(s64, wsk_ref[...], preferred_element_type=F32)
        sv = jnp.dot(s64, wsv_ref[...], preferred_element_type=F32)
        sc = _nt(sq, sk) / math.sqrt(IHD)
        mx = jnp.max(sc, axis=1, keepdims=True)
        p = jnp.exp(sc - mx)
        p = p / jnp.sum(p, axis=1, keepdims=True)
        so = jnp.dot(p, sv, preferred_element_type=F32)
        rr = jax.lax.broadcasted_iota(jnp.int32, (INH, INH * ITOPK), 0)
        cgrp = jax.lax.broadcasted_iota(jnp.int32, (INH, INH * ITOPK), 1) // ITOPK
        A = jnp.where(rr == cgrp, 1.0 / ITOPK, 0.0)
        avg = jnp.dot(A, so, preferred_element_type=F32)
        ib = jnp.dot(avg, wio_ref[...], preferred_element_type=F32)
        bvec = jnp.zeros((1, HID), F32)
        for n in range(NH):
            bvec = bvec + jnp.dot(
                ib[n // 2:n // 2 + 1, :].astype(BF),
                wo_ref[n * DV:(n + 1) * DV, :],
                preferred_element_type=F32)
        bias_ref[...] = bvec

    @pl.when(i > 0)
    def _proj():
        o_ref[...] = jnp.dot(a_ref[...], wo_ref[...],
                             preferred_element_type=F32) + bias_ref[...]


def _kc(gate_t, x16, W_iproj, W_sq, W_sk, W_sv, W_io, attn, W_o):
    S = x16.shape[0]
    const = lambda i: (0, 0)
    return pl.pallas_call(
        _kc_body,
        grid=(1 + S // MB,),
        in_specs=[
            pl.BlockSpec((INH, S), const),
            pl.BlockSpec((S, HID), const),
            pl.BlockSpec((HID, INH * IHD), const),
            pl.BlockSpec((IHD, IHD), const),
            pl.BlockSpec((IHD, IHD), const),
            pl.BlockSpec((IHD, IHD), const),
            pl.BlockSpec((IHD, DV), const),
            pl.BlockSpec((MB, NH * DV), lambda i: (jnp.maximum(i - 1, 0), 0)),
            pl.BlockSpec((NH * DV, HID), const),
        ],
        out_specs=pl.BlockSpec((MB, HID), lambda i: (jnp.maximum(i - 1, 0), 0)),
        out_shape=jax.ShapeDtypeStruct((S, HID), F32),
        scratch_shapes=[pltpu.VMEM((1, HID), F32)],
    )(gate_t, x16, W_iproj, W_sq, W_sk, W_sv, W_io, attn, W_o)


def kernel(x, W_c, W_cp, W_qc, W_qr, W_kc, W_kr, W_v, W_o,
           W_iproj, W_igate, W_sq, W_sk, W_sv, W_iout):
    B, S, _ = x.shape
    x2 = x.reshape(S, HID)
    cos_t, sin_t = _rope_tables(S)

    perm = np.concatenate([np.arange(0, DR, 2), np.arange(1, DR, 2)])
    W_qr_d = W_qr.reshape(DCP, NH, DR)[:, :, perm].reshape(DCP, NH * DR)
    W_kr_d = W_kr[:, perm]

    qc, qr, kc, v, kr, gate_t = _ka(
        x2, W_c.astype(BF), W_cp.astype(BF), W_kr_d.astype(BF), W_igate.T,
        W_qc.astype(BF), W_qr_d.astype(BF), W_kc.astype(BF), W_v.astype(BF),
        cos_t, sin_t)

    attn = _kb(qc, qr, kc, kr, v)

    out = _kc(gate_t, x2.astype(BF), W_iproj.astype(BF), W_sq, W_sk, W_sv,
              W_iout[:, :DV], attn, W_o.astype(BF))
    return out.reshape(B, S, HID)


# speedup vs baseline: 1.4495x; 1.4495x over previous
import math
import numpy as np
import jax
import jax.numpy as jnp
from jax.experimental import pallas as pl
from jax.experimental.pallas import tpu as pltpu

HID = 2048; NH = 16; DK = 128; DR = 64; DV = 128; DC = 512; DCP = 1536
INH = 8; IHD = 128; ITOPK = 8
HALF = DR // 2
MB = 256
QB = 512
HPG = 2
SCALE = 1.0 / math.sqrt(DK + DR)
BF = jnp.bfloat16
F32 = jnp.float32


def _rope_tables(S):
    inv = 1.0 / (10000.0 ** (np.arange(0, DR, 2)[: DR // 2].astype(np.float32) / DR))
    t = np.arange(S, dtype=np.float32)
    f = np.outer(t, inv)
    return jnp.asarray(np.cos(f), dtype=F32), jnp.asarray(np.sin(f), dtype=F32)


def _nt(a, b):
    return jax.lax.dot_general(a, b, (((1,), (1,)), ((), ())),
                               preferred_element_type=F32)


def _ka_body(x_ref, wc_ref, wcp_ref, wkr_ref, wigt_ref, wqc_ref, wqr_ref,
             wkc_ref, wv_ref, cos_ref, sin_ref,
             qc_ref, qr_ref, kc_ref, v_ref, kr_ref, gt_ref):
    xb = x_ref[...]
    xb16 = xb.astype(BF)
    gt_ref[...] = _nt(wigt_ref[...], xb)
    co = cos_ref[...]
    si = sin_ref[...]
    y = jnp.dot(xb16, wkr_ref[...], preferred_element_type=F32)
    a = y[:, :HALF]
    b = y[:, HALF:]
    kr_ref[...] = jnp.concatenate(
        [a * co - b * si, a * si + b * co], axis=1).astype(BF)
    c16 = jnp.dot(xb16, wc_ref[...], preferred_element_type=F32).astype(BF)
    cp16 = jnp.dot(xb16, wcp_ref[...], preferred_element_type=F32).astype(BF)
    qc = jnp.dot(cp16, wqc_ref[...], preferred_element_type=F32) * SCALE
    qc_ref[...] = qc.astype(BF)
    yq = jnp.dot(cp16, wqr_ref[...], preferred_element_type=F32) * SCALE
    parts = []
    for h in range(NH):
        qa = yq[:, h * DR:h * DR + HALF]
        qb = yq[:, h * DR + HALF:(h + 1) * DR]
        parts += [qa * co - qb * si, qa * si + qb * co]
    qr_ref[...] = jnp.concatenate(parts, axis=1).astype(BF)
    kc_ref[...] = jnp.dot(c16, wkc_ref[...], preferred_element_type=F32).astype(BF)
    v_ref[...] = jnp.dot(c16, wv_ref[...], preferred_element_type=F32).astype(BF)


def _ka(x2, W_c, W_cp, W_kr_d, W_igate_t, W_qc, W_qr_d, W_kc, W_v, cos_t, sin_t):
    S = x2.shape[0]
    const = lambda i: (0, 0)
    row = lambda i: (i, 0)
    return pl.pallas_call(
        _ka_body,
        grid=(S // MB,),
        in_specs=[
            pl.BlockSpec((MB, HID), row),
            pl.BlockSpec((HID, DC), const),
            pl.BlockSpec((HID, DCP), const),
            pl.BlockSpec((HID, DR), const),
            pl.BlockSpec((INH, HID), const),
            pl.BlockSpec((DCP, NH * DK), const),
            pl.BlockSpec((DCP, NH * DR), const),
            pl.BlockSpec((DC, NH * DK), const),
            pl.BlockSpec((DC, NH * DV), const),
            pl.BlockSpec((MB, HALF), row),
            pl.BlockSpec((MB, HALF), row),
        ],
        out_specs=[
            pl.BlockSpec((MB, NH * DK), row),
            pl.BlockSpec((MB, NH * DR), row),
            pl.BlockSpec((MB, NH * DK), row),
            pl.BlockSpec((MB, NH * DV), row),
            pl.BlockSpec((MB, DR), row),
            pl.BlockSpec((INH, MB), lambda i: (0, i)),
        ],
        out_shape=[
            jax.ShapeDtypeStruct((S, NH * DK), BF),
            jax.ShapeDtypeStruct((S, NH * DR), BF),
            jax.ShapeDtypeStruct((S, NH * DK), BF),
            jax.ShapeDtypeStruct((S, NH * DV), BF),
            jax.ShapeDtypeStruct((S, DR), BF),
            jax.ShapeDtypeStruct((INH, S), F32),
        ],
    )(x2, W_c, W_cp, W_kr_d, W_igate_t, W_qc, W_qr_d, W_kc, W_v, cos_t, sin_t)


def _kb_body(qc_ref, qr_ref, kc_ref, kr_ref, v_ref, o_ref):
    S = qc_ref.shape[0]
    nq = S // QB
    kr = kr_ref[...]
    for hh in range(HPG):
        kc = kc_ref[:, hh * DK:(hh + 1) * DK]
        v = v_ref[:, hh * DV:(hh + 1) * DV]
        for i in range(nq):
            qc = qc_ref[i * QB:(i + 1) * QB, hh * DK:(hh + 1) * DK]
            qr = qr_ref[i * QB:(i + 1) * QB, hh * DR:(hh + 1) * DR]
            m = jnp.full((QB, 1), -1e30, F32)
            l = jnp.zeros((QB, 1), F32)
            acc = jnp.zeros((QB, DV), F32)
            rows = i * QB + jax.lax.broadcasted_iota(jnp.int32, (QB, QB), 0)
            for j in range(i + 1):
                s = _nt(qc, kc[j * QB:(j + 1) * QB, :])
                s = s + _nt(qr, kr[j * QB:(j + 1) * QB, :])
                if j == i:
                    cols = j * QB + jax.lax.broadcasted_iota(jnp.int32, (QB, QB), 1)
                    s = jnp.where(cols > rows, -1e30, s)
                mb = jnp.max(s, axis=1, keepdims=True)
                m_new = jnp.maximum(m, mb)
                p = jnp.exp(s - m_new)
                corr = jnp.exp(m - m_new)
                l = l * corr + jnp.sum(p, axis=1, keepdims=True)
                acc = acc * corr + jnp.dot(p.astype(BF), v[j * QB:(j + 1) * QB, :],
                                           preferred_element_type=F32)
                m = m_new
            o_ref[i * QB:(i + 1) * QB, hh * DV:(hh + 1) * DV] = (acc / l).astype(BF)


def _kb(qc, qr, kc, kr, v):
    S = qc.shape[0]
    return pl.pallas_call(
        _kb_body,
        grid=(NH // HPG,),
        in_specs=[
            pl.BlockSpec((S, HPG * DK), lambda h: (0, h)),
            pl.BlockSpec((S, HPG * DR), lambda h: (0, h)),
            pl.BlockSpec((S, HPG * DK), lambda h: (0, h)),
            pl.BlockSpec((S, DR), lambda h: (0, 0)),
            pl.BlockSpec((S, HPG * DV), lambda h: (0, h)),
        ],
        out_specs=pl.BlockSpec((S, HPG * DV), lambda h: (0, h)),
        out_shape=jax.ShapeDtypeStruct((S, NH * DV), BF),
    )(qc, qr, kc, kr, v)


def _kc_body(gt_ref, x_ref, wip_ref, wsq_ref, wsk_ref, wsv_ref, wio_ref,
             a_ref, wo_ref, o_ref, bias_ref):
    i = pl.program_id(0)

    @pl.when(i == 0)
    def _indexer():
        g = gt_ref[...]
        Sn = g.shape[1]
        col = jax.lax.broadcasted_iota(jnp.int32, g.shape, 1)
        gg = g
        oh_rows = []
        for _ in range(ITOPK):
            mx = jnp.max(gg, axis=1, keepdims=True)
            amx = jnp.min(jnp.where(gg >= mx, col, Sn), axis=1)
            oh_rows.append((col == amx[:, None]).astype(BF))
            gg = jnp.where(col == amx[:, None], -jnp.inf, gg)
        r64 = jax.lax.broadcasted_iota(jnp.int32, (INH * ITOPK, INH), 0)
        h8 = jax.lax.broadcasted_iota(jnp.int32, (INH * ITOPK, INH), 1)
        oh = jnp.zeros((INH * ITOPK, Sn), F32)
        for t in range(ITOPK):
            pt = jnp.where(r64 == h8 * ITOPK + t, 1.0, 0.0).astype(BF)
            oh = oh + jnp.dot(pt, oh_rows[t], preferred_element_type=F32)
        x_sel = jnp.dot(oh.astype(BF), x_ref[...],
                        preferred_element_type=F32).astype(BF)
        sel = []
        for h in range(INH):
            sel.append(jnp.dot(x_sel[h * ITOPK:(h + 1) * ITOPK, :],
                               wip_ref[:, h * IHD:(h + 1) * IHD],
                               preferred_element_type=F32))
        s64 = jnp.concatenate(sel, axis=0)
        sq = jnp.dot(s64, wsq_ref[...], preferred_element_type=F32)
        sk = jnp.dot(s64, wsk_ref[...], preferred_element_type=F32)
        sv = jnp.dot(s64, wsv_ref[...], preferred_element_type=F32)
        sc = _nt(sq, sk) / math.sqrt(IHD)
        mx = jnp.max(sc, axis=1, keepdims=True)
        p = jnp.exp(sc - mx)
        p = p / jnp.sum(p, axis=1, keepdims=True)
        so = jnp.dot(p, sv, preferred_element_type=F32)
        rr = jax.lax.broadcasted_iota(jnp.int32, (INH, INH * ITOPK), 0)
        cgrp = jax.lax.broadcasted_iota(jnp.int32, (INH, INH * ITOPK), 1) // ITOPK
        A = jnp.where(rr == cgrp, 1.0 / ITOPK, 0.0)
        avg = jnp.dot(A, so, preferred_element_type=F32)
        ib = jnp.dot(avg, wio_ref[...], preferred_element_type=F32)
        bvec = jnp.zeros((1, HID), F32)
        for n in range(NH):
            bvec = bvec + jnp.dot(
                ib[n // 2:n // 2 + 1, :].astype(BF),
                wo_ref[n * DV:(n + 1) * DV, :],
                preferred_element_type=F32)
        bias_ref[...] = bvec

    @pl.when(i > 0)
    def _proj():
        o_ref[...] = jnp.dot(a_ref[...], wo_ref[...],
                             preferred_element_type=F32) + bias_ref[...]


def _kc(gate_t, x16, W_iproj, W_sq, W_sk, W_sv, W_io, attn, W_o):
    S = x16.shape[0]
    const = lambda i: (0, 0)
    return pl.pallas_call(
        _kc_body,
        grid=(1 + S // MB,),
        in_specs=[
            pl.BlockSpec((INH, S), const),
            pl.BlockSpec((S, HID), const),
            pl.BlockSpec((HID, INH * IHD), const),
            pl.BlockSpec((IHD, IHD), const),
            pl.BlockSpec((IHD, IHD), const),
            pl.BlockSpec((IHD, IHD), const),
            pl.BlockSpec((IHD, DV), const),
            pl.BlockSpec((MB, NH * DV), lambda i: (jnp.maximum(i - 1, 0), 0)),
            pl.BlockSpec((NH * DV, HID), const),
        ],
        out_specs=pl.BlockSpec((MB, HID), lambda i: (jnp.maximum(i - 1, 0), 0)),
        out_shape=jax.ShapeDtypeStruct((S, HID), F32),
        scratch_shapes=[pltpu.VMEM((1, HID), F32)],
    )(gate_t, x16, W_iproj, W_sq, W_sk, W_sv, W_io, attn, W_o)


def kernel(x, W_c, W_cp, W_qc, W_qr, W_kc, W_kr, W_v, W_o,
           W_iproj, W_igate, W_sq, W_sk, W_sv, W_iout):
    B, S, _ = x.shape
    x2 = x.reshape(S, HID)
    cos_t, sin_t = _rope_tables(S)

    perm = np.concatenate([np.arange(0, DR, 2), np.arange(1, DR, 2)])
    W_qr_d = W_qr.reshape(DCP, NH, DR)[:, :, perm].reshape(DCP, NH * DR)
    W_kr_d = W_kr[:, perm]

    qc, qr, kc, v, kr, gate_t = _ka(
        x2, W_c.astype(BF), W_cp.astype(BF), W_kr_d.astype(BF), W_igate.T,
        W_qc.astype(BF), W_qr_d.astype(BF), W_kc.astype(BF), W_v.astype(BF),
        cos_t, sin_t)

    attn = _kb(qc, qr, kc, kr, v)

    out = _kc(gate_t, x2.astype(BF), W_iproj.astype(BF), W_sq, W_sk, W_sv,
              W_iout[:, :DV], attn, W_o.astype(BF))
    return out.reshape(B, S, HID)


# speedup vs baseline: 1.4544x; 1.0034x over previous
import math
import numpy as np
import jax
import jax.numpy as jnp
from jax.experimental import pallas as pl
from jax.experimental.pallas import tpu as pltpu

HID = 2048; NH = 16; DK = 128; DR = 64; DV = 128; DC = 512; DCP = 1536
INH = 8; IHD = 128; ITOPK = 8
HALF = DR // 2
MB = 256
QB = 1024
HPG = 2
SCALE = 1.0 / math.sqrt(DK + DR)
BF = jnp.bfloat16
F32 = jnp.float32


def _rope_tables(S):
    inv = 1.0 / (10000.0 ** (np.arange(0, DR, 2)[: DR // 2].astype(np.float32) / DR))
    t = np.arange(S, dtype=np.float32)
    f = np.outer(t, inv)
    return jnp.asarray(np.cos(f), dtype=F32), jnp.asarray(np.sin(f), dtype=F32)


def _nt(a, b):
    return jax.lax.dot_general(a, b, (((1,), (1,)), ((), ())),
                               preferred_element_type=F32)


def _ka_body(x_ref, wc_ref, wcp_ref, wkr_ref, wigt_ref, wqc_ref, wqr_ref,
             wkc_ref, wv_ref, cos_ref, sin_ref,
             qc_ref, qr_ref, kc_ref, v_ref, kr_ref, gt_ref):
    xb = x_ref[...]
    xb16 = xb.astype(BF)
    gt_ref[...] = _nt(wigt_ref[...], xb)
    co = cos_ref[...]
    si = sin_ref[...]
    y = jnp.dot(xb16, wkr_ref[...], preferred_element_type=F32)
    a = y[:, :HALF]
    b = y[:, HALF:]
    kr_ref[...] = jnp.concatenate(
        [a * co - b * si, a * si + b * co], axis=1).astype(BF)
    c16 = jnp.dot(xb16, wc_ref[...], preferred_element_type=F32).astype(BF)
    cp16 = jnp.dot(xb16, wcp_ref[...], preferred_element_type=F32).astype(BF)
    qc = jnp.dot(cp16, wqc_ref[...], preferred_element_type=F32) * SCALE
    qc_ref[...] = qc.astype(BF)
    yq = jnp.dot(cp16, wqr_ref[...], preferred_element_type=F32) * SCALE
    parts = []
    for h in range(NH):
        qa = yq[:, h * DR:h * DR + HALF]
        qb = yq[:, h * DR + HALF:(h + 1) * DR]
        parts += [qa * co - qb * si, qa * si + qb * co]
    qr_ref[...] = jnp.concatenate(parts, axis=1).astype(BF)
    kc_ref[...] = jnp.dot(c16, wkc_ref[...], preferred_element_type=F32).astype(BF)
    v_ref[...] = jnp.dot(c16, wv_ref[...], preferred_element_type=F32).astype(BF)


def _ka(x2, W_c, W_cp, W_kr_d, W_igate_t, W_qc, W_qr_d, W_kc, W_v, cos_t, sin_t):
    S = x2.shape[0]
    const = lambda i: (0, 0)
    row = lambda i: (i, 0)
    return pl.pallas_call(
        _ka_body,
        grid=(S // MB,),
        in_specs=[
            pl.BlockSpec((MB, HID), row),
            pl.BlockSpec((HID, DC), const),
            pl.BlockSpec((HID, DCP), const),
            pl.BlockSpec((HID, DR), const),
            pl.BlockSpec((INH, HID), const),
            pl.BlockSpec((DCP, NH * DK), const),
            pl.BlockSpec((DCP, NH * DR), const),
            pl.BlockSpec((DC, NH * DK), const),
            pl.BlockSpec((DC, NH * DV), const),
            pl.BlockSpec((MB, HALF), row),
            pl.BlockSpec((MB, HALF), row),
        ],
        out_specs=[
            pl.BlockSpec((MB, NH * DK), row),
            pl.BlockSpec((MB, NH * DR), row),
            pl.BlockSpec((MB, NH * DK), row),
            pl.BlockSpec((MB, NH * DV), row),
            pl.BlockSpec((MB, DR), row),
            pl.BlockSpec((INH, MB), lambda i: (0, i)),
        ],
        out_shape=[
            jax.ShapeDtypeStruct((S, NH * DK), BF),
            jax.ShapeDtypeStruct((S, NH * DR), BF),
            jax.ShapeDtypeStruct((S, NH * DK), BF),
            jax.ShapeDtypeStruct((S, NH * DV), BF),
            jax.ShapeDtypeStruct((S, DR), BF),
            jax.ShapeDtypeStruct((INH, S), F32),
        ],
    )(x2, W_c, W_cp, W_kr_d, W_igate_t, W_qc, W_qr_d, W_kc, W_v, cos_t, sin_t)


def _kb_body(qc_ref, qr_ref, kc_ref, kr_ref, v_ref, o_ref):
    S = qc_ref.shape[0]
    nq = S // QB
    kr = kr_ref[...]
    for hh in range(HPG):
        kc = kc_ref[:, hh * DK:(hh + 1) * DK]
        v = v_ref[:, hh * DV:(hh + 1) * DV]
        for i in range(nq):
            qc = qc_ref[i * QB:(i + 1) * QB, hh * DK:(hh + 1) * DK]
            qr = qr_ref[i * QB:(i + 1) * QB, hh * DR:(hh + 1) * DR]
            m = jnp.full((QB, 1), -1e30, F32)
            l = jnp.zeros((QB, 1), F32)
            acc = jnp.zeros((QB, DV), F32)
            rows = i * QB + jax.lax.broadcasted_iota(jnp.int32, (QB, QB), 0)
            for j in range(i + 1):
                s = _nt(qc, kc[j * QB:(j + 1) * QB, :])
                s = s + _nt(qr, kr[j * QB:(j + 1) * QB, :])
                if j == i:
                    cols = j * QB + jax.lax.broadcasted_iota(jnp.int32, (QB, QB), 1)
                    s = jnp.where(cols > rows, -1e30, s)
                mb = jnp.max(s, axis=1, keepdims=True)
                m_new = jnp.maximum(m, mb)
                p = jnp.exp(s - m_new)
                corr = jnp.exp(m - m_new)
                l = l * corr + jnp.sum(p, axis=1, keepdims=True)
                acc = acc * corr + jnp.dot(p.astype(BF), v[j * QB:(j + 1) * QB, :],
                                           preferred_element_type=F32)
                m = m_new
            o_ref[i * QB:(i + 1) * QB, hh * DV:(hh + 1) * DV] = (acc / l).astype(BF)


def _kb(qc, qr, kc, kr, v):
    S = qc.shape[0]
    return pl.pallas_call(
        _kb_body,
        grid=(NH // HPG,),
        in_specs=[
            pl.BlockSpec((S, HPG * DK), lambda h: (0, h)),
            pl.BlockSpec((S, HPG * DR), lambda h: (0, h)),
            pl.BlockSpec((S, HPG * DK), lambda h: (0, h)),
            pl.BlockSpec((S, DR), lambda h: (0, 0)),
            pl.BlockSpec((S, HPG * DV), lambda h: (0, h)),
        ],
        out_specs=pl.BlockSpec((S, HPG * DV), lambda h: (0, h)),
        out_shape=jax.ShapeDtypeStruct((S, NH * DV), BF),
    )(qc, qr, kc, kr, v)


def _kc_body(gt_ref, x_ref, wip_ref, wsq_ref, wsk_ref, wsv_ref, wio_ref,
             a_ref, wo_ref, o_ref, bias_ref):
    i = pl.program_id(0)

    @pl.when(i == 0)
    def _indexer():
        g = gt_ref[...]
        Sn = g.shape[1]
        col = jax.lax.broadcasted_iota(jnp.int32, g.shape, 1)
        gg = g
        oh_rows = []
        for _ in range(ITOPK):
            mx = jnp.max(gg, axis=1, keepdims=True)
            amx = jnp.min(jnp.where(gg >= mx, col, Sn), axis=1)
            oh_rows.append((col == amx[:, None]).astype(BF))
            gg = jnp.where(col == amx[:, None], -jnp.inf, gg)
        r64 = jax.lax.broadcasted_iota(jnp.int32, (INH * ITOPK, INH), 0)
        h8 = jax.lax.broadcasted_iota(jnp.int32, (INH * ITOPK, INH), 1)
        oh = jnp.zeros((INH * ITOPK, Sn), F32)
        for t in range(ITOPK):
            pt = jnp.where(r64 == h8 * ITOPK + t, 1.0, 0.0).astype(BF)
            oh = oh + jnp.dot(pt, oh_rows[t], preferred_element_type=F32)
        x_sel = jnp.dot(oh.astype(BF), x_ref[...],
                        preferred_element_type=F32).astype(BF)
        sel = []
        for h in range(INH):
            sel.append(jnp.dot(x_sel[h * ITOPK:(h + 1) * ITOPK, :],
                               wip_ref[:, h * IHD:(h + 1) * IHD],
                               preferred_element_type=F32))
        s64 = jnp.concatenate(sel, axis=0)
        sq = jnp.dot(s64, wsq_ref[...], preferred_element_type=F32)
        sk = jnp.dot(s64, wsk_ref[...], preferred_element_type=F32)
        sv = jnp.dot(s64, wsv_ref[...], preferred_element_type=F32)
        sc = _nt(sq, sk) / math.sqrt(IHD)
        mx = jnp.max(sc, axis=1, keepdims=True)
        p = jnp.exp(sc - mx)
        p = p / jnp.sum(p, axis=1, keepdims=True)
        so = jnp.dot(p, sv, preferred_element_type=F32)
        rr = jax.lax.broadcasted_iota(jnp.int32, (INH, INH * ITOPK), 0)
        cgrp = jax.lax.broadcasted_iota(jnp.int32, (INH, INH * ITOPK), 1) // ITOPK
        A = jnp.where(rr == cgrp, 1.0 / ITOPK, 0.0)
        avg = jnp.dot(A, so, preferred_element_type=F32)
        ib = jnp.dot(avg, wio_ref[...], preferred_element_type=F32)
        bvec = jnp.zeros((1, HID), F32)
        for n in range(NH):
            bvec = bvec + jnp.dot(
                ib[n // 2:n // 2 + 1, :].astype(BF),
                wo_ref[n * DV:(n + 1) * DV, :],
                preferred_element_type=F32)
        bias_ref[...] = bvec

    @pl.when(i > 0)
    def _proj():
        o_ref[...] = jnp.dot(a_ref[...], wo_ref[...],
                             preferred_element_type=F32) + bias_ref[...]


def _kc(gate_t, x16, W_iproj, W_sq, W_sk, W_sv, W_io, attn, W_o):
    S = x16.shape[0]
    const = lambda i: (0, 0)
    return pl.pallas_call(
        _kc_body,
        grid=(1 + S // MB,),
        in_specs=[
            pl.BlockSpec((INH, S), const),
            pl.BlockSpec((S, HID), const),
            pl.BlockSpec((HID, INH * IHD), const),
            pl.BlockSpec((IHD, IHD), const),
            pl.BlockSpec((IHD, IHD), const),
            pl.BlockSpec((IHD, IHD), const),
            pl.BlockSpec((IHD, DV), const),
            pl.BlockSpec((MB, NH * DV), lambda i: (jnp.maximum(i - 1, 0), 0)),
            pl.BlockSpec((NH * DV, HID), const),
        ],
        out_specs=pl.BlockSpec((MB, HID), lambda i: (jnp.maximum(i - 1, 0), 0)),
        out_shape=jax.ShapeDtypeStruct((S, HID), F32),
        scratch_shapes=[pltpu.VMEM((1, HID), F32)],
    )(gate_t, x16, W_iproj, W_sq, W_sk, W_sv, W_io, attn, W_o)


def kernel(x, W_c, W_cp, W_qc, W_qr, W_kc, W_kr, W_v, W_o,
           W_iproj, W_igate, W_sq, W_sk, W_sv, W_iout):
    B, S, _ = x.shape
    x2 = x.reshape(S, HID)
    cos_t, sin_t = _rope_tables(S)

    perm = np.concatenate([np.arange(0, DR, 2), np.arange(1, DR, 2)])
    W_qr_d = W_qr.reshape(DCP, NH, DR)[:, :, perm].reshape(DCP, NH * DR)
    W_kr_d = W_kr[:, perm]

    qc, qr, kc, v, kr, gate_t = _ka(
        x2, W_c.astype(BF), W_cp.astype(BF), W_kr_d.astype(BF), W_igate.T,
        W_qc.astype(BF), W_qr_d.astype(BF), W_kc.astype(BF), W_v.astype(BF),
        cos_t, sin_t)

    attn = _kb(qc, qr, kc, kr, v)

    out = _kc(gate_t, x2.astype(BF), W_iproj.astype(BF), W_sq, W_sk, W_sv,
              W_iout[:, :DV], attn, W_o.astype(BF))
    return out.reshape(B, S, HID)
